# baseline (speedup 1.0000x reference)
import functools

import jax
import jax.numpy as jnp
from jax import lax
from jax.experimental import pallas as pl
from jax.experimental.pallas import tpu as pltpu
from jax.experimental.pallas import tpu_sc as plsc

D = 1024
F = 2048
E = 8
EPAD = 128
K = 2
N = 4096
P = N * K
BLK_A = 512
BLK_G = 256
G = P // BLK_G + E
MAX_ROWS = G * BLK_G

NC, NS, L = 2, 16, 16
NW = NC * NS

SPIKE_THRESHOLD = 0.1
EPSILON = 1e-8


def _gate_body(x_ref, wg_ref, bg_ref, noise_ref,
               xn_ref, idx_ref, wts_ref, aux_ref, acc_ref):
    x = x_ref[...]
    scores = jnp.mean(x, axis=1, keepdims=True)
    spiked = jnp.where(scores > SPIKE_THRESHOLD, x, 0.0)
    denom = jnp.sum(spiked, axis=1, keepdims=True) + EPSILON
    xn = spiked / denom
    xn_ref[...] = xn.astype(jnp.bfloat16)

    logits = jnp.dot(xn, wg_ref[...], preferred_element_type=jnp.float32)
    logits = logits + bg_ref[...] + noise_ref[...]
    lane = lax.broadcasted_iota(jnp.int32, logits.shape, 1)

    m1 = jnp.max(logits, axis=1, keepdims=True)
    i1 = jnp.min(jnp.where(logits == m1, lane, EPAD), axis=1, keepdims=True)
    l2 = jnp.where(lane == i1, -3e38, logits)
    m2 = jnp.max(l2, axis=1, keepdims=True)
    i2 = jnp.min(jnp.where(l2 == m2, lane, EPAD), axis=1, keepdims=True)

    masked = jnp.where(logits >= m2, logits, -1e9)
    ex = jnp.exp(masked - m1)
    z = jnp.sum(ex, axis=1, keepdims=True)
    w1 = 1.0 / z
    w2 = jnp.exp(m2 - m1) / z

    idx_ref[...] = jnp.where(lane == 0, i1, jnp.where(lane == 1, i2, 0))
    wts_ref[...] = jnp.where(lane == 0, w1, jnp.where(lane == 1, w2, 0.0))

    g = pl.program_id(0)

    @pl.when(g == 0)
    def _():
        acc_ref[...] = jnp.zeros_like(acc_ref)

    acc_ref[...] += jnp.sum(ex / z, axis=0, keepdims=True)

    @pl.when(g == pl.num_programs(0) - 1)
    def _():
        usage = acc_ref[...]
        lane8 = lax.broadcasted_iota(jnp.int32, usage.shape, 1) < E
        total = jnp.sum(usage)
        imp = jnp.where(lane8, usage / (total + 1e-10), 0.0)
        mean = jnp.sum(imp) / E
        var = jnp.sum(jnp.where(lane8, (imp - mean) ** 2, 0.0)) / E
        aux = jnp.sqrt(var) / (mean + 1e-10)
        aux_ref[...] = jnp.where(
            lax.broadcasted_iota(jnp.int32, usage.shape, 1) == 0, aux, 0.0)


def _gate_call(x2, wg_pad, bg_pad, noise_pad):
    nblk = N // BLK_A
    return pl.pallas_call(
        _gate_body,
        grid=(nblk,),
        in_specs=[
            pl.BlockSpec((BLK_A, D), lambda g: (g, 0)),
            pl.BlockSpec((D, EPAD), lambda g: (0, 0)),
            pl.BlockSpec((1, EPAD), lambda g: (0, 0)),
            pl.BlockSpec((BLK_A, EPAD), lambda g: (g, 0)),
        ],
        out_specs=[
            pl.BlockSpec((BLK_A, D), lambda g: (g, 0)),
            pl.BlockSpec((BLK_A, EPAD), lambda g: (g, 0)),
            pl.BlockSpec((BLK_A, EPAD), lambda g: (g, 0)),
            pl.BlockSpec((1, EPAD), lambda g: (0, 0)),
        ],
        out_shape=[
            jax.ShapeDtypeStruct((N, D), jnp.bfloat16),
            jax.ShapeDtypeStruct((N, EPAD), jnp.int32),
            jax.ShapeDtypeStruct((N, EPAD), jnp.float32),
            jax.ShapeDtypeStruct((1, EPAD), jnp.float32),
        ],
        scratch_shapes=[pltpu.VMEM((1, EPAD), jnp.float32)],
        compiler_params=pltpu.CompilerParams(
            dimension_semantics=("arbitrary",)),
    )(x2, wg_pad, bg_pad, noise_pad)


def _gmm_body(be_ref, xp_ref, w1_ref, b1_ref, w2_ref, b2_ref, ws_ref, out_ref):
    @pl.when(pl.program_id(0) < be_ref[G])
    def _():
        h = jnp.dot(xp_ref[...], w1_ref[0],
                    preferred_element_type=jnp.float32)
        h = h + b1_ref[0]
        h = h * (1.0 / (1.0 + jnp.exp(-h)))
        o = jnp.dot(h.astype(jnp.bfloat16), w2_ref[0],
                    preferred_element_type=jnp.float32)
        o = o + b2_ref[0]
        out_ref[...] = o * ws_ref[...]


def _gmm_call(be, xp, W1, b1, W2, b2, ws):
    grid_spec = pltpu.PrefetchScalarGridSpec(
        num_scalar_prefetch=1,
        grid=(G,),
        in_specs=[
            pl.BlockSpec((BLK_G, D), lambda g, be: (g, 0)),
            pl.BlockSpec((1, D, F), lambda g, be: (be[g], 0, 0)),
            pl.BlockSpec((1, 1, F), lambda g, be: (be[g], 0, 0)),
            pl.BlockSpec((1, F, D), lambda g, be: (be[g], 0, 0)),
            pl.BlockSpec((1, 1, D), lambda g, be: (be[g], 0, 0)),
            pl.BlockSpec((BLK_G, 1), lambda g, be: (g, 0)),
        ],
        out_specs=pl.BlockSpec((BLK_G, D), lambda g, be: (g, 0)),
    )
    return pl.pallas_call(
        _gmm_body,
        grid_spec=grid_spec,
        out_shape=jax.ShapeDtypeStruct((MAX_ROWS, D), jnp.float32),
        compiler_params=pltpu.CompilerParams(
            dimension_semantics=("arbitrary",)),
    )(be, xp, W1, b1, W2, b2, ws)


_G_PER_W = MAX_ROWS // NW
_G_CH = 64
_G_NCH = _G_PER_W // _G_CH
_DW = D // 2


def _sc_gather_body(xn_hbm, idx_hbm, out_hbm, idx_v, rows_a, rows_b,
                    gsem_a, gsem_b, wsem_a, wsem_b):
    wid = lax.axis_index("s") * NC + lax.axis_index("c")
    base = wid * _G_PER_W
    pltpu.sync_copy(idx_hbm.at[wid], idx_v)
    bufs = (rows_a, rows_b)
    gsems = (gsem_a, gsem_b)
    wsems = (wsem_a, wsem_b)

    def start_gather(c):
        b = c % 2
        return pltpu.async_copy(xn_hbm.at[idx_v.at[c]], bufs[b], gsems[b])

    gcopy = {0: start_gather(0)}
    wcopy = {}
    for c in range(_G_NCH):
        b = c % 2
        gcopy.pop(c).wait()
        if c + 1 < _G_NCH:
            if c - 1 in wcopy:
                wcopy.pop(c - 1).wait()
            gcopy[c + 1] = start_gather(c + 1)
        wcopy[c] = pltpu.async_copy(
            bufs[b], out_hbm.at[pl.ds(base + c * _G_CH, _G_CH)], wsems[b])
    for c in sorted(wcopy):
        wcopy[c].wait()


def _sc_gather(xn, idx3):
    return pl.kernel(
        _sc_gather_body,
        out_type=jax.ShapeDtypeStruct((MAX_ROWS, _DW), jnp.int32),
        mesh=plsc.VectorSubcoreMesh(
            core_axis_name="c", subcore_axis_name="s",
            num_cores=NC, num_subcores=NS),
        scratch_types=[
            pltpu.VMEM((_G_NCH, _G_CH), jnp.int32),
            pltpu.VMEM((_G_CH, _DW), jnp.int32),
            pltpu.VMEM((_G_CH, _DW), jnp.int32),
            pltpu.SemaphoreType.DMA,
            pltpu.SemaphoreType.DMA,
            pltpu.SemaphoreType.DMA,
            pltpu.SemaphoreType.DMA,
        ],
    )(xn, idx3)


_C_TOK_W = N // NW
_C_TCH = 16
_C_NCH = _C_TOK_W // _C_TCH


def _sc_combine_body(og_hbm, slot_hbm, out_hbm, idx_v, rows_a, rows_b,
                     acc_a, acc_b, gsem_a, gsem_b, wsem_a, wsem_b):
    wid = lax.axis_index("s") * NC + lax.axis_index("c")
    tbase = wid * _C_TOK_W
    pltpu.sync_copy(slot_hbm.at[wid], idx_v)
    rbufs = (rows_a, rows_b)
    abufs = (acc_a, acc_b)
    gsems = (gsem_a, gsem_b)
    wsems = (wsem_a, wsem_b)

    def start_gather(c):
        b = c % 2
        return pltpu.async_copy(og_hbm.at[idx_v.at[c]], rbufs[b], gsems[b])

    gcopy = {0: start_gather(0)}
    wcopy = {}
    for c in range(_C_NCH):
        b = c % 2
        gcopy.pop(c).wait()
        if c + 1 < _C_NCH:
            gcopy[c + 1] = start_gather(c + 1)
        if c - 2 in wcopy:
            wcopy.pop(c - 2).wait()
        rows = rbufs[b]
        acc = abufs[b]

        def body(t, carry):
            for j in range(D // L):
                s = pl.ds(j * L, L)
                acc[t, s] = rows[2 * t, s] + rows[2 * t + 1, s]
            return carry

        lax.fori_loop(0, _C_TCH, body, 0)
        wcopy[c] = pltpu.async_copy(
            acc, out_hbm.at[pl.ds(tbase + c * _C_TCH, _C_TCH)], wsems[b])
    for c in sorted(wcopy):
        wcopy[c].wait()


def _sc_combine(og, slot3):
    return pl.kernel(
        _sc_combine_body,
        out_type=jax.ShapeDtypeStruct((N, D), jnp.float32),
        mesh=plsc.VectorSubcoreMesh(
            core_axis_name="c", subcore_axis_name="s",
            num_cores=NC, num_subcores=NS),
        scratch_types=[
            pltpu.VMEM((_C_NCH, 2 * _C_TCH), jnp.int32),
            pltpu.VMEM((2 * _C_TCH, D), jnp.float32),
            pltpu.VMEM((2 * _C_TCH, D), jnp.float32),
            pltpu.VMEM((_C_TCH, D), jnp.float32),
            pltpu.VMEM((_C_TCH, D), jnp.float32),
            pltpu.SemaphoreType.DMA,
            pltpu.SemaphoreType.DMA,
            pltpu.SemaphoreType.DMA,
            pltpu.SemaphoreType.DMA,
        ],
    )(og, slot3)


def _routing_meta(top2, wts2):
    i32 = jnp.int32
    e_flat = top2.reshape(-1)
    order = jnp.argsort(e_flat, stable=True)
    sorted_e = e_flat[order]
    counts = jnp.bincount(e_flat, length=E)
    pc = ((counts + BLK_G - 1) // BLK_G) * BLK_G
    po = jnp.concatenate([jnp.zeros(1, pc.dtype), jnp.cumsum(pc)[:-1]])
    uo = jnp.concatenate([jnp.zeros(1, counts.dtype), jnp.cumsum(counts)[:-1]])
    slot_sorted = (po[sorted_e] + (jnp.arange(P) - uo[sorted_e])).astype(i32)
    slot_of = jnp.zeros((P,), i32).at[order].set(slot_sorted)
    tok_for_slot = jnp.zeros((MAX_ROWS,), i32).at[slot_sorted].set(
        (order // K).astype(i32))
    w_slot = jnp.zeros((MAX_ROWS, 1), jnp.float32).at[slot_sorted, 0].set(
        wts2.reshape(-1)[order])
    pe = jnp.cumsum(pc)
    be = jnp.minimum(
        jnp.searchsorted(pe, jnp.arange(G) * BLK_G, side="right"),
        E - 1).astype(i32)
    n_active = (pe[-1] // BLK_G).astype(i32)
    be = jnp.concatenate([be, n_active[None]])
    return slot_of, tok_for_slot, w_slot, be


def kernel(x, W_gate, b_gate, W1, b1, W2, b2):
    x2 = jnp.asarray(x, jnp.float32).reshape(N, D)

    wg_pad = jnp.zeros((D, EPAD), jnp.float32).at[:, :E].set(W_gate)
    bg_pad = jnp.full((1, EPAD), -1e30, jnp.float32).at[0, :E].set(b_gate)
    noise = jax.random.normal(jax.random.key(42), (2, 2048, E)) * 0.01
    noise_pad = jnp.zeros((N, EPAD), jnp.float32).at[:, :E].set(
        noise.reshape(N, E))

    xn, idx128, wts128, aux128 = _gate_call(x2, wg_pad, bg_pad, noise_pad)
    top2 = idx128[:, :K]
    wts2 = wts128[:, :K]
    aux_loss = aux128[0, 0]

    slot_of, tok_for_slot, w_slot, be = _routing_meta(top2, wts2)

    xn_i32 = lax.bitcast_convert_type(xn.reshape(N, _DW, 2), jnp.int32)
    xp_i32 = _sc_gather(xn_i32, tok_for_slot.reshape(NW, _G_NCH, _G_CH))
    xp = lax.bitcast_convert_type(xp_i32, jnp.bfloat16).reshape(MAX_ROWS, D)
    og = _gmm_call(be, xp,
                   W1.astype(jnp.bfloat16), b1.reshape(E, 1, F),
                   W2.astype(jnp.bfloat16), b2.reshape(E, 1, D), w_slot)
    out = _sc_combine(og, slot_of.reshape(NW, _C_NCH, 2 * _C_TCH))

    return (out.reshape(2, 2048, D), top2.reshape(2, 2048, K), aux_loss)


# speedup vs baseline: 1.7834x; 1.7834x over previous
import functools

import jax
import jax.numpy as jnp
from jax import lax
from jax.experimental import pallas as pl
from jax.experimental.pallas import tpu as pltpu
from jax.experimental.pallas import tpu_sc as plsc

D = 1024
F = 2048
E = 8
EPAD = 128
K = 2
N = 4096
P = N * K
BLK_A = 512
BLK_G = 256
G = P // BLK_G + E
MAX_ROWS = G * BLK_G

NC, NS, L = 2, 16, 16
NW = NC * NS

SPIKE_THRESHOLD = 0.1
EPSILON = 1e-8


def _gate_body(x_ref, wg_ref, bg_ref, noise_ref,
               xn_ref, idx_ref, wts_ref, aux_ref, acc_ref):
    x = x_ref[...]
    scores = jnp.mean(x, axis=1, keepdims=True)
    spiked = jnp.where(scores > SPIKE_THRESHOLD, x, 0.0)
    denom = jnp.sum(spiked, axis=1, keepdims=True) + EPSILON
    xn = spiked / denom
    xn_ref[...] = xn

    logits = jnp.dot(xn, wg_ref[...], preferred_element_type=jnp.float32)
    logits = logits + bg_ref[...] + noise_ref[...]
    lane = lax.broadcasted_iota(jnp.int32, logits.shape, 1)

    m1 = jnp.max(logits, axis=1, keepdims=True)
    i1 = jnp.min(jnp.where(logits == m1, lane, EPAD), axis=1, keepdims=True)
    l2 = jnp.where(lane == i1, -3e38, logits)
    m2 = jnp.max(l2, axis=1, keepdims=True)
    i2 = jnp.min(jnp.where(l2 == m2, lane, EPAD), axis=1, keepdims=True)

    masked = jnp.where(logits >= m2, logits, -1e9)
    ex = jnp.exp(masked - m1)
    z = jnp.sum(ex, axis=1, keepdims=True)
    w1 = 1.0 / z
    w2 = jnp.exp(m2 - m1) / z

    idx_ref[...] = jnp.where(lane == 0, i1, jnp.where(lane == 1, i2, 0))
    wts_ref[...] = jnp.where(lane == 0, w1, jnp.where(lane == 1, w2, 0.0))

    g = pl.program_id(0)

    @pl.when(g == 0)
    def _():
        acc_ref[...] = jnp.zeros_like(acc_ref)

    acc_ref[...] += jnp.sum(ex / z, axis=0, keepdims=True)

    @pl.when(g == pl.num_programs(0) - 1)
    def _():
        usage = acc_ref[...]
        lane8 = lax.broadcasted_iota(jnp.int32, usage.shape, 1) < E
        total = jnp.sum(usage)
        imp = jnp.where(lane8, usage / (total + 1e-10), 0.0)
        mean = jnp.sum(imp) / E
        var = jnp.sum(jnp.where(lane8, (imp - mean) ** 2, 0.0)) / E
        aux = jnp.sqrt(var) / (mean + 1e-10)
        aux_ref[...] = jnp.where(
            lax.broadcasted_iota(jnp.int32, usage.shape, 1) == 0, aux, 0.0)


def _gate_call(x2, wg_pad, bg_pad, noise_pad):
    nblk = N // BLK_A
    return pl.pallas_call(
        _gate_body,
        grid=(nblk,),
        in_specs=[
            pl.BlockSpec((BLK_A, D), lambda g: (g, 0)),
            pl.BlockSpec((D, EPAD), lambda g: (0, 0)),
            pl.BlockSpec((1, EPAD), lambda g: (0, 0)),
            pl.BlockSpec((BLK_A, EPAD), lambda g: (g, 0)),
        ],
        out_specs=[
            pl.BlockSpec((BLK_A, D), lambda g: (g, 0)),
            pl.BlockSpec((BLK_A, EPAD), lambda g: (g, 0)),
            pl.BlockSpec((BLK_A, EPAD), lambda g: (g, 0)),
            pl.BlockSpec((1, EPAD), lambda g: (0, 0)),
        ],
        out_shape=[
            jax.ShapeDtypeStruct((N, D), jnp.float32),
            jax.ShapeDtypeStruct((N, EPAD), jnp.int32),
            jax.ShapeDtypeStruct((N, EPAD), jnp.float32),
            jax.ShapeDtypeStruct((1, EPAD), jnp.float32),
        ],
        scratch_shapes=[pltpu.VMEM((1, EPAD), jnp.float32)],
        compiler_params=pltpu.CompilerParams(
            dimension_semantics=("arbitrary",)),
    )(x2, wg_pad, bg_pad, noise_pad)


def _gmm_body(be_ref, xp_ref, w1_ref, b1_ref, w2_ref, b2_ref, ws_ref, out_ref):
    @pl.when(pl.program_id(0) < be_ref[G])
    def _():
        x_bf = xp_ref[...].astype(jnp.bfloat16)
        h = jnp.dot(x_bf, w1_ref[0].astype(jnp.bfloat16),
                    preferred_element_type=jnp.float32)
        h = h + b1_ref[0]
        h = h * (1.0 / (1.0 + jnp.exp(-h)))
        o = jnp.dot(h.astype(jnp.bfloat16), w2_ref[0].astype(jnp.bfloat16),
                    preferred_element_type=jnp.float32)
        o = o + b2_ref[0]
        out_ref[...] = o * ws_ref[...]


def _gmm_call(be, xp, W1, b1, W2, b2, ws):
    grid_spec = pltpu.PrefetchScalarGridSpec(
        num_scalar_prefetch=1,
        grid=(G,),
        in_specs=[
            pl.BlockSpec((BLK_G, D), lambda g, be: (g, 0)),
            pl.BlockSpec((1, D, F), lambda g, be: (be[g], 0, 0)),
            pl.BlockSpec((1, 1, F), lambda g, be: (be[g], 0, 0)),
            pl.BlockSpec((1, F, D), lambda g, be: (be[g], 0, 0)),
            pl.BlockSpec((1, 1, D), lambda g, be: (be[g], 0, 0)),
            pl.BlockSpec((BLK_G, 1), lambda g, be: (g, 0)),
        ],
        out_specs=pl.BlockSpec((BLK_G, D), lambda g, be: (g, 0)),
    )
    return pl.pallas_call(
        _gmm_body,
        grid_spec=grid_spec,
        out_shape=jax.ShapeDtypeStruct((MAX_ROWS, D), jnp.float32),
        compiler_params=pltpu.CompilerParams(
            dimension_semantics=("arbitrary",)),
    )(be, xp, W1, b1, W2, b2, ws)


_G_PER_W = MAX_ROWS // NW
_G_CH = 40
_G_NCH = _G_PER_W // _G_CH


def _sc_gather_body(xn_hbm, idx_hbm, out_hbm, idx_v, rows_a, rows_b,
                    gsem_a, gsem_b, wsem_a, wsem_b):
    wid = lax.axis_index("s") * NC + lax.axis_index("c")
    base = wid * _G_PER_W
    pltpu.sync_copy(idx_hbm.at[wid], idx_v)
    bufs = (rows_a, rows_b)
    gsems = (gsem_a, gsem_b)
    wsems = (wsem_a, wsem_b)

    def start_gather(c):
        b = c % 2
        return pltpu.async_copy(xn_hbm.at[idx_v.at[c]], bufs[b], gsems[b])

    gcopy = {0: start_gather(0)}
    wcopy = {}
    for c in range(_G_NCH):
        b = c % 2
        gcopy.pop(c).wait()
        if c + 1 < _G_NCH:
            if c - 1 in wcopy:
                wcopy.pop(c - 1).wait()
            gcopy[c + 1] = start_gather(c + 1)
        wcopy[c] = pltpu.async_copy(
            bufs[b], out_hbm.at[pl.ds(base + c * _G_CH, _G_CH)], wsems[b])
    for c in sorted(wcopy):
        wcopy[c].wait()


def _sc_gather(xn, idx3):
    return pl.kernel(
        _sc_gather_body,
        out_type=jax.ShapeDtypeStruct((MAX_ROWS, D), jnp.float32),
        mesh=plsc.VectorSubcoreMesh(
            core_axis_name="c", subcore_axis_name="s",
            num_cores=NC, num_subcores=NS),
        scratch_types=[
            pltpu.VMEM((_G_NCH, _G_CH), jnp.int32),
            pltpu.VMEM((_G_CH, D), jnp.float32),
            pltpu.VMEM((_G_CH, D), jnp.float32),
            pltpu.SemaphoreType.DMA,
            pltpu.SemaphoreType.DMA,
            pltpu.SemaphoreType.DMA,
            pltpu.SemaphoreType.DMA,
        ],
    )(xn, idx3)


_C_TOK_W = N // NW
_C_TCH = 16
_C_NCH = _C_TOK_W // _C_TCH


def _sc_combine_body(og_hbm, slot_hbm, out_hbm, idx_v, rows_a, rows_b,
                     acc_a, acc_b, gsem_a, gsem_b, wsem_a, wsem_b):
    wid = lax.axis_index("s") * NC + lax.axis_index("c")
    tbase = wid * _C_TOK_W
    pltpu.sync_copy(slot_hbm.at[wid], idx_v)
    rbufs = (rows_a, rows_b)
    abufs = (acc_a, acc_b)
    gsems = (gsem_a, gsem_b)
    wsems = (wsem_a, wsem_b)

    def start_gather(c):
        b = c % 2
        return pltpu.async_copy(og_hbm.at[idx_v.at[c]], rbufs[b], gsems[b])

    gcopy = {0: start_gather(0)}
    wcopy = {}
    for c in range(_C_NCH):
        b = c % 2
        gcopy.pop(c).wait()
        if c + 1 < _C_NCH:
            gcopy[c + 1] = start_gather(c + 1)
        if c - 2 in wcopy:
            wcopy.pop(c - 2).wait()
        rows = rbufs[b]
        acc = abufs[b]

        def body(t, carry):
            for j in range(D // L):
                s = pl.ds(j * L, L)
                acc[t, s] = rows[2 * t, s] + rows[2 * t + 1, s]
            return carry

        lax.fori_loop(0, _C_TCH, body, 0)
        wcopy[c] = pltpu.async_copy(
            acc, out_hbm.at[pl.ds(tbase + c * _C_TCH, _C_TCH)], wsems[b])
    for c in sorted(wcopy):
        wcopy[c].wait()


def _sc_combine(og, slot3):
    return pl.kernel(
        _sc_combine_body,
        out_type=jax.ShapeDtypeStruct((N, D), jnp.float32),
        mesh=plsc.VectorSubcoreMesh(
            core_axis_name="c", subcore_axis_name="s",
            num_cores=NC, num_subcores=NS),
        scratch_types=[
            pltpu.VMEM((_C_NCH, 2 * _C_TCH), jnp.int32),
            pltpu.VMEM((2 * _C_TCH, D), jnp.float32),
            pltpu.VMEM((2 * _C_TCH, D), jnp.float32),
            pltpu.VMEM((_C_TCH, D), jnp.float32),
            pltpu.VMEM((_C_TCH, D), jnp.float32),
            pltpu.SemaphoreType.DMA,
            pltpu.SemaphoreType.DMA,
            pltpu.SemaphoreType.DMA,
            pltpu.SemaphoreType.DMA,
        ],
    )(og, slot3)


def _routing_meta(top2, wts2):
    i32 = jnp.int32
    e_flat = top2.reshape(-1)
    order = jnp.argsort(e_flat, stable=True)
    sorted_e = e_flat[order]
    counts = jnp.bincount(e_flat, length=E)
    pc = ((counts + BLK_G - 1) // BLK_G) * BLK_G
    po = jnp.concatenate([jnp.zeros(1, pc.dtype), jnp.cumsum(pc)[:-1]])
    uo = jnp.concatenate([jnp.zeros(1, counts.dtype), jnp.cumsum(counts)[:-1]])
    slot_sorted = (po[sorted_e] + (jnp.arange(P) - uo[sorted_e])).astype(i32)
    slot_of = jnp.zeros((P,), i32).at[order].set(slot_sorted)
    tok_for_slot = jnp.zeros((MAX_ROWS,), i32).at[slot_sorted].set(
        (order // K).astype(i32))
    w_slot = jnp.zeros((MAX_ROWS, 1), jnp.float32).at[slot_sorted, 0].set(
        wts2.reshape(-1)[order])
    pe = jnp.cumsum(pc)
    be = jnp.minimum(
        jnp.searchsorted(pe, jnp.arange(G) * BLK_G, side="right"),
        E - 1).astype(i32)
    n_active = (pe[-1] // BLK_G).astype(i32)
    be = jnp.concatenate([be, n_active[None]])
    return slot_of, tok_for_slot, w_slot, be


def kernel(x, W_gate, b_gate, W1, b1, W2, b2):
    x2 = jnp.asarray(x, jnp.float32).reshape(N, D)

    wg_pad = jnp.zeros((D, EPAD), jnp.float32).at[:, :E].set(W_gate)
    bg_pad = jnp.full((1, EPAD), -1e30, jnp.float32).at[0, :E].set(b_gate)
    noise = jax.random.normal(jax.random.key(42), (2, 2048, E)) * 0.01
    noise_pad = jnp.zeros((N, EPAD), jnp.float32).at[:, :E].set(
        noise.reshape(N, E))

    xn, idx128, wts128, aux128 = _gate_call(x2, wg_pad, bg_pad, noise_pad)
    top2 = idx128[:, :K]
    wts2 = wts128[:, :K]
    aux_loss = aux128[0, 0]

    slot_of, tok_for_slot, w_slot, be = _routing_meta(top2, wts2)

    xp = _sc_gather(xn, tok_for_slot.reshape(NW, _G_NCH, _G_CH))
    og = _gmm_call(be, xp, W1, b1.reshape(E, 1, F), W2, b2.reshape(E, 1, D),
                   w_slot)
    out = _sc_combine(og, slot_of.reshape(NW, _C_NCH, 2 * _C_TCH))

    return (out.reshape(2, 2048, D), top2.reshape(2, 2048, K), aux_loss)


# speedup vs baseline: 1.8649x; 1.0457x over previous
import functools

import jax
import jax.numpy as jnp
from jax import lax
from jax.experimental import pallas as pl
from jax.experimental.pallas import tpu as pltpu
from jax.experimental.pallas import tpu_sc as plsc

D = 1024
F = 2048
E = 8
EPAD = 128
K = 2
N = 4096
P = N * K
BLK_A = 512
BLK_G = 256
G = P // BLK_G + E
MAX_ROWS = G * BLK_G

NC, NS, L = 2, 16, 16
NW = NC * NS

SPIKE_THRESHOLD = 0.1
EPSILON = 1e-8


def _gate_body(x_ref, wg_ref, bg_ref, noise_ref,
               xn_ref, idx_ref, wts_ref, aux_ref, acc_ref):
    x = x_ref[...]
    scores = jnp.mean(x, axis=1, keepdims=True)
    spiked = jnp.where(scores > SPIKE_THRESHOLD, x, 0.0)
    denom = jnp.sum(spiked, axis=1, keepdims=True) + EPSILON
    xn = spiked / denom
    xn_ref[...] = xn

    logits = jnp.dot(xn, wg_ref[...], preferred_element_type=jnp.float32)
    logits = logits + bg_ref[...] + noise_ref[...]
    lane = lax.broadcasted_iota(jnp.int32, logits.shape, 1)

    m1 = jnp.max(logits, axis=1, keepdims=True)
    i1 = jnp.min(jnp.where(logits == m1, lane, EPAD), axis=1, keepdims=True)
    l2 = jnp.where(lane == i1, -3e38, logits)
    m2 = jnp.max(l2, axis=1, keepdims=True)
    i2 = jnp.min(jnp.where(l2 == m2, lane, EPAD), axis=1, keepdims=True)

    masked = jnp.where(logits >= m2, logits, -1e9)
    ex = jnp.exp(masked - m1)
    z = jnp.sum(ex, axis=1, keepdims=True)
    w1 = 1.0 / z
    w2 = jnp.exp(m2 - m1) / z

    idx_ref[...] = jnp.where(lane == 0, i1, jnp.where(lane == 1, i2, 0))
    wts_ref[...] = jnp.where(lane == 0, w1, jnp.where(lane == 1, w2, 0.0))

    g = pl.program_id(0)

    @pl.when(g == 0)
    def _():
        acc_ref[...] = jnp.zeros_like(acc_ref)

    acc_ref[...] += jnp.sum(ex / z, axis=0, keepdims=True)

    @pl.when(g == pl.num_programs(0) - 1)
    def _():
        usage = acc_ref[...]
        lane8 = lax.broadcasted_iota(jnp.int32, usage.shape, 1) < E
        total = jnp.sum(usage)
        imp = jnp.where(lane8, usage / (total + 1e-10), 0.0)
        mean = jnp.sum(imp) / E
        var = jnp.sum(jnp.where(lane8, (imp - mean) ** 2, 0.0)) / E
        aux = jnp.sqrt(var) / (mean + 1e-10)
        aux_ref[...] = jnp.where(
            lax.broadcasted_iota(jnp.int32, usage.shape, 1) == 0, aux, 0.0)


def _gate_call(x2, wg_pad, bg_pad, noise_pad):
    nblk = N // BLK_A
    return pl.pallas_call(
        _gate_body,
        grid=(nblk,),
        in_specs=[
            pl.BlockSpec((BLK_A, D), lambda g: (g, 0)),
            pl.BlockSpec((D, EPAD), lambda g: (0, 0)),
            pl.BlockSpec((1, EPAD), lambda g: (0, 0)),
            pl.BlockSpec((BLK_A, EPAD), lambda g: (g, 0)),
        ],
        out_specs=[
            pl.BlockSpec((BLK_A, D), lambda g: (g, 0)),
            pl.BlockSpec((BLK_A, EPAD), lambda g: (g, 0)),
            pl.BlockSpec((BLK_A, EPAD), lambda g: (g, 0)),
            pl.BlockSpec((1, EPAD), lambda g: (0, 0)),
        ],
        out_shape=[
            jax.ShapeDtypeStruct((N, D), jnp.float32),
            jax.ShapeDtypeStruct((N, EPAD), jnp.int32),
            jax.ShapeDtypeStruct((N, EPAD), jnp.float32),
            jax.ShapeDtypeStruct((1, EPAD), jnp.float32),
        ],
        scratch_shapes=[pltpu.VMEM((1, EPAD), jnp.float32)],
        compiler_params=pltpu.CompilerParams(
            dimension_semantics=("arbitrary",)),
    )(x2, wg_pad, bg_pad, noise_pad)


def _gmm_body(be_ref, xp_ref, w1_ref, b1_ref, w2_ref, b2_ref, ws_ref, out_ref):
    @pl.when(pl.program_id(0) < be_ref[G])
    def _():
        x_bf = xp_ref[...].astype(jnp.bfloat16)
        h = jnp.dot(x_bf, w1_ref[0].astype(jnp.bfloat16),
                    preferred_element_type=jnp.float32)
        h = h + b1_ref[0]
        h = h * (1.0 / (1.0 + jnp.exp(-h)))
        o = jnp.dot(h.astype(jnp.bfloat16), w2_ref[0].astype(jnp.bfloat16),
                    preferred_element_type=jnp.float32)
        o = o + b2_ref[0]
        out_ref[...] = o * ws_ref[...]


def _gmm_call(be, xp, W1, b1, W2, b2, ws):
    grid_spec = pltpu.PrefetchScalarGridSpec(
        num_scalar_prefetch=1,
        grid=(G,),
        in_specs=[
            pl.BlockSpec((BLK_G, D), lambda g, be: (g, 0)),
            pl.BlockSpec((1, D, F), lambda g, be: (be[g], 0, 0)),
            pl.BlockSpec((1, 1, F), lambda g, be: (be[g], 0, 0)),
            pl.BlockSpec((1, F, D), lambda g, be: (be[g], 0, 0)),
            pl.BlockSpec((1, 1, D), lambda g, be: (be[g], 0, 0)),
            pl.BlockSpec((BLK_G, 1), lambda g, be: (g, 0)),
        ],
        out_specs=pl.BlockSpec((BLK_G, D), lambda g, be: (g, 0)),
    )
    return pl.pallas_call(
        _gmm_body,
        grid_spec=grid_spec,
        out_shape=jax.ShapeDtypeStruct((MAX_ROWS, D), jnp.float32),
        compiler_params=pltpu.CompilerParams(
            dimension_semantics=("arbitrary",)),
    )(be, xp, W1, b1, W2, b2, ws)


_G_PER_W = MAX_ROWS // NW
_G_CH = 40
_G_NCH = _G_PER_W // _G_CH


def _sc_gather_body(xn_hbm, idx_hbm, out_hbm, idx_v, rows_a, rows_b,
                    gsem_a, gsem_b, wsem_a, wsem_b):
    wid = lax.axis_index("s") * NC + lax.axis_index("c")
    base = wid * _G_PER_W
    pltpu.sync_copy(idx_hbm.at[wid], idx_v)
    bufs = (rows_a, rows_b)
    gsems = (gsem_a, gsem_b)
    wsems = (wsem_a, wsem_b)

    def start_gather(c):
        b = c % 2
        return pltpu.async_copy(xn_hbm.at[idx_v.at[c]], bufs[b], gsems[b])

    gcopy = {0: start_gather(0)}
    wcopy = {}
    for c in range(_G_NCH):
        b = c % 2
        gcopy.pop(c).wait()
        if c + 1 < _G_NCH:
            if c - 1 in wcopy:
                wcopy.pop(c - 1).wait()
            gcopy[c + 1] = start_gather(c + 1)
        wcopy[c] = pltpu.async_copy(
            bufs[b], out_hbm.at[pl.ds(base + c * _G_CH, _G_CH)], wsems[b])
    for c in sorted(wcopy):
        wcopy[c].wait()


def _sc_gather(xn, idx3):
    return pl.kernel(
        _sc_gather_body,
        out_type=jax.ShapeDtypeStruct((MAX_ROWS, D), jnp.float32),
        mesh=plsc.VectorSubcoreMesh(
            core_axis_name="c", subcore_axis_name="s",
            num_cores=NC, num_subcores=NS),
        scratch_types=[
            pltpu.VMEM((_G_NCH, _G_CH), jnp.int32),
            pltpu.VMEM((_G_CH, D), jnp.float32),
            pltpu.VMEM((_G_CH, D), jnp.float32),
            pltpu.SemaphoreType.DMA,
            pltpu.SemaphoreType.DMA,
            pltpu.SemaphoreType.DMA,
            pltpu.SemaphoreType.DMA,
        ],
    )(xn, idx3)


_C_TOK_W = N // NW
_C_TCH = 16
_C_NCH = _C_TOK_W // _C_TCH


def _sc_combine_body(og_hbm, slot_hbm, out_hbm, idx_v, rows_a, rows_b,
                     acc_a, acc_b, gsem_a, gsem_b, wsem_a, wsem_b):
    wid = lax.axis_index("s") * NC + lax.axis_index("c")
    tbase = wid * _C_TOK_W
    pltpu.sync_copy(slot_hbm.at[wid], idx_v)
    rbufs = (rows_a, rows_b)
    abufs = (acc_a, acc_b)
    gsems = (gsem_a, gsem_b)
    wsems = (wsem_a, wsem_b)

    def start_gather(c):
        b = c % 2
        return pltpu.async_copy(og_hbm.at[idx_v.at[c]], rbufs[b], gsems[b])

    gcopy = {0: start_gather(0)}
    wcopy = {}
    for c in range(_C_NCH):
        b = c % 2
        gcopy.pop(c).wait()
        if c + 1 < _C_NCH:
            gcopy[c + 1] = start_gather(c + 1)
        if c - 2 in wcopy:
            wcopy.pop(c - 2).wait()
        rows = rbufs[b]
        acc = abufs[b]

        def body(t, carry):
            for j in range(D // L):
                s = pl.ds(j * L, L)
                acc[t, s] = rows[2 * t, s] + rows[2 * t + 1, s]
            return carry

        lax.fori_loop(0, _C_TCH, body, 0)
        wcopy[c] = pltpu.async_copy(
            acc, out_hbm.at[pl.ds(tbase + c * _C_TCH, _C_TCH)], wsems[b])
    for c in sorted(wcopy):
        wcopy[c].wait()


def _sc_combine(og, slot3):
    return pl.kernel(
        _sc_combine_body,
        out_type=jax.ShapeDtypeStruct((N, D), jnp.float32),
        mesh=plsc.VectorSubcoreMesh(
            core_axis_name="c", subcore_axis_name="s",
            num_cores=NC, num_subcores=NS),
        scratch_types=[
            pltpu.VMEM((_C_NCH, 2 * _C_TCH), jnp.int32),
            pltpu.VMEM((2 * _C_TCH, D), jnp.float32),
            pltpu.VMEM((2 * _C_TCH, D), jnp.float32),
            pltpu.VMEM((_C_TCH, D), jnp.float32),
            pltpu.VMEM((_C_TCH, D), jnp.float32),
            pltpu.SemaphoreType.DMA,
            pltpu.SemaphoreType.DMA,
            pltpu.SemaphoreType.DMA,
            pltpu.SemaphoreType.DMA,
        ],
    )(og, slot3)


def _routing_meta(top2, wts2):
    i32 = jnp.int32
    e_flat = top2.reshape(-1)
    oh = (e_flat[:, None] == jnp.arange(E)[None, :]).astype(i32)
    csum = jnp.cumsum(oh, axis=0)
    counts = csum[-1]
    rank = jnp.take_along_axis(csum, e_flat[:, None], 1)[:, 0] - 1
    pc = ((counts + BLK_G - 1) // BLK_G) * BLK_G
    pe = jnp.cumsum(pc)
    po = pe - pc
    slot_of = (po[e_flat] + rank).astype(i32)
    tok_for_slot = jnp.zeros((MAX_ROWS,), i32).at[slot_of].set(
        (jnp.arange(P, dtype=i32) // K))
    w_slot = jnp.zeros((MAX_ROWS, 1), jnp.float32).at[slot_of, 0].set(
        wts2.reshape(-1))
    be = jnp.minimum(
        jnp.sum((jnp.arange(G)[:, None] * BLK_G >= pe[None, :]).astype(i32),
                axis=1),
        E - 1).astype(i32)
    n_active = (pe[-1] // BLK_G).astype(i32)
    be = jnp.concatenate([be, n_active[None]])
    return slot_of, tok_for_slot, w_slot, be


def kernel(x, W_gate, b_gate, W1, b1, W2, b2):
    x2 = jnp.asarray(x, jnp.float32).reshape(N, D)

    wg_pad = jnp.zeros((D, EPAD), jnp.float32).at[:, :E].set(W_gate)
    bg_pad = jnp.full((1, EPAD), -1e30, jnp.float32).at[0, :E].set(b_gate)
    noise = jax.random.normal(jax.random.key(42), (2, 2048, E)) * 0.01
    noise_pad = jnp.zeros((N, EPAD), jnp.float32).at[:, :E].set(
        noise.reshape(N, E))

    xn, idx128, wts128, aux128 = _gate_call(x2, wg_pad, bg_pad, noise_pad)
    top2 = idx128[:, :K]
    wts2 = wts128[:, :K]
    aux_loss = aux128[0, 0]

    slot_of, tok_for_slot, w_slot, be = _routing_meta(top2, wts2)

    xp = _sc_gather(xn, tok_for_slot.reshape(NW, _G_NCH, _G_CH))
    og = _gmm_call(be, xp, W1, b1.reshape(E, 1, F), W2, b2.reshape(E, 1, D),
                   w_slot)
    out = _sc_combine(og, slot_of.reshape(NW, _C_NCH, 2 * _C_TCH))

    return (out.reshape(2, 2048, D), top2.reshape(2, 2048, K), aux_loss)


# speedup vs baseline: 2.5610x; 1.3732x over previous
import functools

import jax
import jax.numpy as jnp
from jax import lax
from jax.experimental import pallas as pl
from jax.experimental.pallas import tpu as pltpu
from jax.experimental.pallas import tpu_sc as plsc

D = 1024
F = 2048
E = 8
EPAD = 128
K = 2
N = 4096
P = N * K
BLK_A = 512
BLK_G = 256
G = P // BLK_G + E
MAX_ROWS = G * BLK_G

NC, NS, L = 2, 16, 16
NW = NC * NS

SPIKE_THRESHOLD = 0.1
EPSILON = 1e-8


def _gate_body(x_ref, wg_ref, bg_ref, noise_ref,
               xn_ref, idx_ref, wts_ref, aux_ref, acc_ref):
    x = x_ref[...]
    scores = jnp.mean(x, axis=1, keepdims=True)
    spiked = jnp.where(scores > SPIKE_THRESHOLD, x, 0.0)
    denom = jnp.sum(spiked, axis=1, keepdims=True) + EPSILON
    xn = spiked / denom
    xn_ref[...] = xn

    logits = jnp.dot(xn, wg_ref[...], preferred_element_type=jnp.float32)
    logits = logits + bg_ref[...] + noise_ref[...]
    lane = lax.broadcasted_iota(jnp.int32, logits.shape, 1)

    m1 = jnp.max(logits, axis=1, keepdims=True)
    i1 = jnp.min(jnp.where(logits == m1, lane, EPAD), axis=1, keepdims=True)
    l2 = jnp.where(lane == i1, -3e38, logits)
    m2 = jnp.max(l2, axis=1, keepdims=True)
    i2 = jnp.min(jnp.where(l2 == m2, lane, EPAD), axis=1, keepdims=True)

    masked = jnp.where(logits >= m2, logits, -1e9)
    ex = jnp.exp(masked - m1)
    z = jnp.sum(ex, axis=1, keepdims=True)
    w1 = 1.0 / z
    w2 = jnp.exp(m2 - m1) / z

    idx_ref[...] = jnp.where(lane == 0, i1, jnp.where(lane == 1, i2, 0))
    wts_ref[...] = jnp.where(lane == 0, w1, jnp.where(lane == 1, w2, 0.0))

    g = pl.program_id(0)

    @pl.when(g == 0)
    def _():
        acc_ref[...] = jnp.zeros_like(acc_ref)

    acc_ref[...] += jnp.sum(ex / z, axis=0, keepdims=True)

    @pl.when(g == pl.num_programs(0) - 1)
    def _():
        usage = acc_ref[...]
        lane8 = lax.broadcasted_iota(jnp.int32, usage.shape, 1) < E
        total = jnp.sum(usage)
        imp = jnp.where(lane8, usage / (total + 1e-10), 0.0)
        mean = jnp.sum(imp) / E
        var = jnp.sum(jnp.where(lane8, (imp - mean) ** 2, 0.0)) / E
        aux = jnp.sqrt(var) / (mean + 1e-10)
        aux_ref[...] = jnp.where(
            lax.broadcasted_iota(jnp.int32, usage.shape, 1) == 0, aux, 0.0)


def _gate_call(x2, wg_pad, bg_pad, noise_pad):
    nblk = N // BLK_A
    return pl.pallas_call(
        _gate_body,
        grid=(nblk,),
        in_specs=[
            pl.BlockSpec((BLK_A, D), lambda g: (g, 0)),
            pl.BlockSpec((D, EPAD), lambda g: (0, 0)),
            pl.BlockSpec((1, EPAD), lambda g: (0, 0)),
            pl.BlockSpec((BLK_A, EPAD), lambda g: (g, 0)),
        ],
        out_specs=[
            pl.BlockSpec((BLK_A, D), lambda g: (g, 0)),
            pl.BlockSpec((BLK_A, EPAD), lambda g: (g, 0)),
            pl.BlockSpec((BLK_A, EPAD), lambda g: (g, 0)),
            pl.BlockSpec((1, EPAD), lambda g: (0, 0)),
        ],
        out_shape=[
            jax.ShapeDtypeStruct((N, D), jnp.float32),
            jax.ShapeDtypeStruct((N, EPAD), jnp.int32),
            jax.ShapeDtypeStruct((N, EPAD), jnp.float32),
            jax.ShapeDtypeStruct((1, EPAD), jnp.float32),
        ],
        scratch_shapes=[pltpu.VMEM((1, EPAD), jnp.float32)],
        compiler_params=pltpu.CompilerParams(
            dimension_semantics=("arbitrary",)),
    )(x2, wg_pad, bg_pad, noise_pad)


def _gmm_body(be_ref, xp_ref, w1_ref, b1_ref, w2_ref, b2_ref, ws_ref, out_ref):
    @pl.when(pl.program_id(0) < be_ref[G])
    def _():
        x_bf = xp_ref[...].astype(jnp.bfloat16)
        h = jnp.dot(x_bf, w1_ref[0].astype(jnp.bfloat16),
                    preferred_element_type=jnp.float32)
        h = h + b1_ref[0]
        h = h * (1.0 / (1.0 + jnp.exp(-h)))
        o = jnp.dot(h.astype(jnp.bfloat16), w2_ref[0].astype(jnp.bfloat16),
                    preferred_element_type=jnp.float32)
        o = o + b2_ref[0]
        out_ref[...] = o * ws_ref[...]


def _gmm_call(be, xp, W1, b1, W2, b2, ws):
    grid_spec = pltpu.PrefetchScalarGridSpec(
        num_scalar_prefetch=1,
        grid=(G,),
        in_specs=[
            pl.BlockSpec((BLK_G, D), lambda g, be: (g, 0)),
            pl.BlockSpec((1, D, F), lambda g, be: (be[g], 0, 0)),
            pl.BlockSpec((1, 1, F), lambda g, be: (be[g], 0, 0)),
            pl.BlockSpec((1, F, D), lambda g, be: (be[g], 0, 0)),
            pl.BlockSpec((1, 1, D), lambda g, be: (be[g], 0, 0)),
            pl.BlockSpec((BLK_G, 1), lambda g, be: (g, 0)),
        ],
        out_specs=pl.BlockSpec((BLK_G, D), lambda g, be: (g, 0)),
    )
    return pl.pallas_call(
        _gmm_body,
        grid_spec=grid_spec,
        out_shape=jax.ShapeDtypeStruct((MAX_ROWS, D), jnp.float32),
        compiler_params=pltpu.CompilerParams(
            dimension_semantics=("arbitrary",)),
    )(be, xp, W1, b1, W2, b2, ws)


_D_TOK_W = N // NW
_D_TCH = 32
_D_NCH = _D_TOK_W // _D_TCH


def _sc_dispatch_body(xn_hbm, slot_hbm, out_hbm, idx_v, buf_a, buf_b,
                      rs_a, rs_b, ws_a0, ws_a1, ws_b0, ws_b1):
    wid = lax.axis_index("s") * NC + lax.axis_index("c")
    tokbase = wid * _D_TOK_W
    pltpu.sync_copy(slot_hbm.at[wid], idx_v)
    bufs = (buf_a, buf_b)
    rsems = (rs_a, rs_b)
    wsems = ((ws_a0, ws_a1), (ws_b0, ws_b1))

    def start_read(c):
        b = c % 2
        return pltpu.async_copy(
            xn_hbm.at[pl.ds(tokbase + c * _D_TCH, _D_TCH)], bufs[b], rsems[b])

    rcopy = {0: start_read(0)}
    wcopy = {}
    for c in range(_D_NCH):
        b = c % 2
        rcopy.pop(c).wait()
        if c + 1 < _D_NCH:
            if c - 1 in wcopy:
                for h in wcopy.pop(c - 1):
                    h.wait()
            rcopy[c + 1] = start_read(c + 1)
        wcopy[c] = (
            pltpu.async_copy(bufs[b], out_hbm.at[idx_v.at[2 * c]],
                             wsems[b][0]),
            pltpu.async_copy(bufs[b], out_hbm.at[idx_v.at[2 * c + 1]],
                             wsems[b][1]),
        )
    for c in sorted(wcopy):
        for h in wcopy[c]:
            h.wait()


def _sc_dispatch(xn, slot3):
    return pl.kernel(
        _sc_dispatch_body,
        out_type=jax.ShapeDtypeStruct((MAX_ROWS, D), jnp.float32),
        mesh=plsc.VectorSubcoreMesh(
            core_axis_name="c", subcore_axis_name="s",
            num_cores=NC, num_subcores=NS),
        scratch_types=[
            pltpu.VMEM((2 * _D_NCH, _D_TCH), jnp.int32),
            pltpu.VMEM((_D_TCH, D), jnp.float32),
            pltpu.VMEM((_D_TCH, D), jnp.float32),
            pltpu.SemaphoreType.DMA,
            pltpu.SemaphoreType.DMA,
            pltpu.SemaphoreType.DMA,
            pltpu.SemaphoreType.DMA,
            pltpu.SemaphoreType.DMA,
            pltpu.SemaphoreType.DMA,
        ],
    )(xn, slot3)


_C_TOK_W = N // NW
_C_TCH = 16
_C_NCH = _C_TOK_W // _C_TCH


def _sc_combine_body(og_hbm, slot_hbm, out_hbm, idx_v, rows_a, rows_b,
                     acc_a, acc_b, gsem_a, gsem_b, wsem_a, wsem_b):
    wid = lax.axis_index("s") * NC + lax.axis_index("c")
    tbase = wid * _C_TOK_W
    pltpu.sync_copy(slot_hbm.at[wid], idx_v)
    rbufs = (rows_a, rows_b)
    abufs = (acc_a, acc_b)
    gsems = (gsem_a, gsem_b)
    wsems = (wsem_a, wsem_b)

    def start_gather(c):
        b = c % 2
        return pltpu.async_copy(og_hbm.at[idx_v.at[c]], rbufs[b], gsems[b])

    gcopy = {0: start_gather(0)}
    wcopy = {}
    for c in range(_C_NCH):
        b = c % 2
        gcopy.pop(c).wait()
        if c + 1 < _C_NCH:
            gcopy[c + 1] = start_gather(c + 1)
        if c - 2 in wcopy:
            wcopy.pop(c - 2).wait()
        rows = rbufs[b]
        acc = abufs[b]

        def body(t, carry):
            for j in range(D // L):
                s = pl.ds(j * L, L)
                acc[t, s] = rows[2 * t, s] + rows[2 * t + 1, s]
            return carry

        lax.fori_loop(0, _C_TCH, body, 0)
        wcopy[c] = pltpu.async_copy(
            acc, out_hbm.at[pl.ds(tbase + c * _C_TCH, _C_TCH)], wsems[b])
    for c in sorted(wcopy):
        wcopy[c].wait()


def _sc_combine(og, slot3):
    return pl.kernel(
        _sc_combine_body,
        out_type=jax.ShapeDtypeStruct((N, D), jnp.float32),
        mesh=plsc.VectorSubcoreMesh(
            core_axis_name="c", subcore_axis_name="s",
            num_cores=NC, num_subcores=NS),
        scratch_types=[
            pltpu.VMEM((_C_NCH, 2 * _C_TCH), jnp.int32),
            pltpu.VMEM((2 * _C_TCH, D), jnp.float32),
            pltpu.VMEM((2 * _C_TCH, D), jnp.float32),
            pltpu.VMEM((_C_TCH, D), jnp.float32),
            pltpu.VMEM((_C_TCH, D), jnp.float32),
            pltpu.SemaphoreType.DMA,
            pltpu.SemaphoreType.DMA,
            pltpu.SemaphoreType.DMA,
            pltpu.SemaphoreType.DMA,
        ],
    )(og, slot3)


def _routing_meta(top2, wts2):
    i32 = jnp.int32
    e_flat = top2.reshape(-1)
    oh = (e_flat[:, None] == jnp.arange(E)[None, :]).astype(i32)
    csum = jnp.cumsum(oh, axis=0)
    counts = csum[-1]
    rank = jnp.take_along_axis(csum, e_flat[:, None], 1)[:, 0] - 1
    pc = ((counts + BLK_G - 1) // BLK_G) * BLK_G
    pe = jnp.cumsum(pc)
    po = pe - pc
    slot_of = (po[e_flat] + rank).astype(i32)
    w_slot = jnp.zeros((MAX_ROWS, 1), jnp.float32).at[slot_of, 0].set(
        wts2.reshape(-1))
    be = jnp.minimum(
        jnp.sum((jnp.arange(G)[:, None] * BLK_G >= pe[None, :]).astype(i32),
                axis=1),
        E - 1).astype(i32)
    n_active = (pe[-1] // BLK_G).astype(i32)
    be = jnp.concatenate([be, n_active[None]])
    return slot_of, w_slot, be


def kernel(x, W_gate, b_gate, W1, b1, W2, b2):
    x2 = jnp.asarray(x, jnp.float32).reshape(N, D)

    wg_pad = jnp.zeros((D, EPAD), jnp.float32).at[:, :E].set(W_gate)
    bg_pad = jnp.full((1, EPAD), -1e30, jnp.float32).at[0, :E].set(b_gate)
    noise = jax.random.normal(jax.random.key(42), (2, 2048, E)) * 0.01
    noise_pad = jnp.zeros((N, EPAD), jnp.float32).at[:, :E].set(
        noise.reshape(N, E))

    xn, idx128, wts128, aux128 = _gate_call(x2, wg_pad, bg_pad, noise_pad)
    top2 = idx128[:, :K]
    wts2 = wts128[:, :K]
    aux_loss = aux128[0, 0]

    slot_of, w_slot, be = _routing_meta(top2, wts2)

    disp_slots = (slot_of.reshape(NW, _D_NCH, _D_TCH, K)
                  .transpose(0, 1, 3, 2).reshape(NW, 2 * _D_NCH, _D_TCH))
    xp = _sc_dispatch(xn, disp_slots)
    og = _gmm_call(be, xp, W1, b1.reshape(E, 1, F), W2, b2.reshape(E, 1, D),
                   w_slot)
    out = _sc_combine(og, slot_of.reshape(NW, _C_NCH, 2 * _C_TCH))

    return (out.reshape(2, 2048, D), top2.reshape(2, 2048, K), aux_loss)


# speedup vs baseline: 2.6789x; 1.0460x over previous
import functools

import jax
import jax.numpy as jnp
from jax import lax
from jax.experimental import pallas as pl
from jax.experimental.pallas import tpu as pltpu
from jax.experimental.pallas import tpu_sc as plsc

D = 1024
F = 2048
E = 8
EPAD = 128
K = 2
N = 4096
P = N * K
BLK_A = 512
BLK_G = 256
G = P // BLK_G + E
MAX_ROWS = G * BLK_G

NC, NS, L = 2, 16, 16
NW = NC * NS

SPIKE_THRESHOLD = 0.1
EPSILON = 1e-8


def _gate_body(x_ref, wg_ref, bg_ref, noise_ref,
               xn_ref, idx_ref, wts_ref, aux_ref, acc_ref):
    x = x_ref[...]
    scores = jnp.mean(x, axis=1, keepdims=True)
    spiked = jnp.where(scores > SPIKE_THRESHOLD, x, 0.0)
    denom = jnp.sum(spiked, axis=1, keepdims=True) + EPSILON
    xn = spiked / denom
    xn_ref[...] = xn

    logits = jnp.dot(xn, wg_ref[...], preferred_element_type=jnp.float32)
    logits = logits + bg_ref[...] + noise_ref[...]
    lane = lax.broadcasted_iota(jnp.int32, logits.shape, 1)

    m1 = jnp.max(logits, axis=1, keepdims=True)
    i1 = jnp.min(jnp.where(logits == m1, lane, EPAD), axis=1, keepdims=True)
    l2 = jnp.where(lane == i1, -3e38, logits)
    m2 = jnp.max(l2, axis=1, keepdims=True)
    i2 = jnp.min(jnp.where(l2 == m2, lane, EPAD), axis=1, keepdims=True)

    masked = jnp.where(logits >= m2, logits, -1e9)
    ex = jnp.exp(masked - m1)
    z = jnp.sum(ex, axis=1, keepdims=True)
    w1 = 1.0 / z
    w2 = jnp.exp(m2 - m1) / z

    idx_ref[...] = jnp.where(lane == 0, i1, jnp.where(lane == 1, i2, 0))
    wts_ref[...] = jnp.where(lane == 0, w1, jnp.where(lane == 1, w2, 0.0))

    g = pl.program_id(0)

    @pl.when(g == 0)
    def _():
        acc_ref[...] = jnp.zeros_like(acc_ref)

    acc_ref[...] += jnp.sum(ex / z, axis=0, keepdims=True)

    @pl.when(g == pl.num_programs(0) - 1)
    def _():
        usage = acc_ref[...]
        lane8 = lax.broadcasted_iota(jnp.int32, usage.shape, 1) < E
        total = jnp.sum(usage)
        imp = jnp.where(lane8, usage / (total + 1e-10), 0.0)
        mean = jnp.sum(imp) / E
        var = jnp.sum(jnp.where(lane8, (imp - mean) ** 2, 0.0)) / E
        aux = jnp.sqrt(var) / (mean + 1e-10)
        aux_ref[...] = jnp.where(
            lax.broadcasted_iota(jnp.int32, usage.shape, 1) == 0, aux, 0.0)


def _gate_call(x2, wg_pad, bg_pad, noise_pad):
    nblk = N // BLK_A
    return pl.pallas_call(
        _gate_body,
        grid=(nblk,),
        in_specs=[
            pl.BlockSpec((BLK_A, D), lambda g: (g, 0)),
            pl.BlockSpec((D, EPAD), lambda g: (0, 0)),
            pl.BlockSpec((1, EPAD), lambda g: (0, 0)),
            pl.BlockSpec((BLK_A, EPAD), lambda g: (g, 0)),
        ],
        out_specs=[
            pl.BlockSpec((BLK_A, D), lambda g: (g, 0)),
            pl.BlockSpec((BLK_A, EPAD), lambda g: (g, 0)),
            pl.BlockSpec((BLK_A, EPAD), lambda g: (g, 0)),
            pl.BlockSpec((1, EPAD), lambda g: (0, 0)),
        ],
        out_shape=[
            jax.ShapeDtypeStruct((N, D), jnp.float32),
            jax.ShapeDtypeStruct((N, EPAD), jnp.int32),
            jax.ShapeDtypeStruct((N, EPAD), jnp.float32),
            jax.ShapeDtypeStruct((1, EPAD), jnp.float32),
        ],
        scratch_shapes=[pltpu.VMEM((1, EPAD), jnp.float32)],
        compiler_params=pltpu.CompilerParams(
            dimension_semantics=("arbitrary",)),
    )(x2, wg_pad, bg_pad, noise_pad)


def _gmm_body(be_ref, xp_ref, w1_ref, b1_ref, w2_ref, b2_ref, out_ref):
    @pl.when(pl.program_id(0) < be_ref[G])
    def _():
        x_bf = xp_ref[...].astype(jnp.bfloat16)
        h = jnp.dot(x_bf, w1_ref[0].astype(jnp.bfloat16),
                    preferred_element_type=jnp.float32)
        h = h + b1_ref[0]
        h = h * (1.0 / (1.0 + jnp.exp(-h)))
        o = jnp.dot(h.astype(jnp.bfloat16), w2_ref[0].astype(jnp.bfloat16),
                    preferred_element_type=jnp.float32)
        out_ref[...] = o + b2_ref[0]


def _gmm_call(be, xp, W1, b1, W2, b2):
    grid_spec = pltpu.PrefetchScalarGridSpec(
        num_scalar_prefetch=1,
        grid=(G,),
        in_specs=[
            pl.BlockSpec((BLK_G, D), lambda g, be: (g, 0)),
            pl.BlockSpec((1, D, F), lambda g, be: (be[g], 0, 0)),
            pl.BlockSpec((1, 1, F), lambda g, be: (be[g], 0, 0)),
            pl.BlockSpec((1, F, D), lambda g, be: (be[g], 0, 0)),
            pl.BlockSpec((1, 1, D), lambda g, be: (be[g], 0, 0)),
        ],
        out_specs=pl.BlockSpec((BLK_G, D), lambda g, be: (g, 0)),
    )
    return pl.pallas_call(
        _gmm_body,
        grid_spec=grid_spec,
        out_shape=jax.ShapeDtypeStruct((MAX_ROWS, D), jnp.float32),
        compiler_params=pltpu.CompilerParams(
            dimension_semantics=("arbitrary",)),
    )(be, xp, W1, b1, W2, b2)


_D_TOK_W = N // NW
_D_TCH = 32
_D_NCH = _D_TOK_W // _D_TCH


def _sc_dispatch_body(xn_hbm, slot_hbm, out_hbm, idx_v, buf_a, buf_b,
                      rs_a, rs_b, ws_a0, ws_a1, ws_b0, ws_b1):
    wid = lax.axis_index("s") * NC + lax.axis_index("c")
    tokbase = wid * _D_TOK_W
    pltpu.sync_copy(slot_hbm.at[wid], idx_v)
    bufs = (buf_a, buf_b)
    rsems = (rs_a, rs_b)
    wsems = ((ws_a0, ws_a1), (ws_b0, ws_b1))

    def start_read(c):
        b = c % 2
        return pltpu.async_copy(
            xn_hbm.at[pl.ds(tokbase + c * _D_TCH, _D_TCH)], bufs[b], rsems[b])

    rcopy = {0: start_read(0)}
    wcopy = {}
    for c in range(_D_NCH):
        b = c % 2
        rcopy.pop(c).wait()
        if c + 1 < _D_NCH:
            if c - 1 in wcopy:
                for h in wcopy.pop(c - 1):
                    h.wait()
            rcopy[c + 1] = start_read(c + 1)
        wcopy[c] = (
            pltpu.async_copy(bufs[b], out_hbm.at[idx_v.at[2 * c]],
                             wsems[b][0]),
            pltpu.async_copy(bufs[b], out_hbm.at[idx_v.at[2 * c + 1]],
                             wsems[b][1]),
        )
    for c in sorted(wcopy):
        for h in wcopy[c]:
            h.wait()


def _sc_dispatch(xn, slot3):
    return pl.kernel(
        _sc_dispatch_body,
        out_type=jax.ShapeDtypeStruct((MAX_ROWS, D), jnp.float32),
        mesh=plsc.VectorSubcoreMesh(
            core_axis_name="c", subcore_axis_name="s",
            num_cores=NC, num_subcores=NS),
        scratch_types=[
            pltpu.VMEM((2 * _D_NCH, _D_TCH), jnp.int32),
            pltpu.VMEM((_D_TCH, D), jnp.float32),
            pltpu.VMEM((_D_TCH, D), jnp.float32),
            pltpu.SemaphoreType.DMA,
            pltpu.SemaphoreType.DMA,
            pltpu.SemaphoreType.DMA,
            pltpu.SemaphoreType.DMA,
            pltpu.SemaphoreType.DMA,
            pltpu.SemaphoreType.DMA,
        ],
    )(xn, slot3)


_C_TOK_W = N // NW
_C_TCH = 16
_C_NCH = _C_TOK_W // _C_TCH


def _sc_combine_body(og_hbm, slot_hbm, wts_hbm, out_hbm, idx_v, w_v,
                     rows_a, rows_b, acc_a, acc_b,
                     gsem_a, gsem_b, wsem_a, wsem_b):
    wid = lax.axis_index("s") * NC + lax.axis_index("c")
    tbase = wid * _C_TOK_W
    pltpu.sync_copy(slot_hbm.at[wid], idx_v)
    pltpu.sync_copy(wts_hbm.at[wid], w_v)
    rbufs = (rows_a, rows_b)
    abufs = (acc_a, acc_b)
    gsems = (gsem_a, gsem_b)
    wsems = (wsem_a, wsem_b)

    def start_gather(c):
        b = c % 2
        return pltpu.async_copy(og_hbm.at[idx_v.at[c]], rbufs[b], gsems[b])

    gcopy = {0: start_gather(0)}
    wcopy = {}
    for c in range(_C_NCH):
        b = c % 2
        gcopy.pop(c).wait()
        if c + 1 < _C_NCH:
            gcopy[c + 1] = start_gather(c + 1)
        if c - 2 in wcopy:
            wcopy.pop(c - 2).wait()
        rows = rbufs[b]
        acc = abufs[b]
        wv0 = w_v[c, 0, :]
        wv1 = w_v[c, 1, :]

        def body(t, carry):
            bidx = lax.broadcasted_iota(jnp.int32, (L,), 0) * 0 + t
            w0 = jnp.take_along_axis(wv0, bidx, axis=0)
            w1 = jnp.take_along_axis(wv1, bidx, axis=0)
            for j in range(D // L):
                s = pl.ds(j * L, L)
                acc[t, s] = w0 * rows[2 * t, s] + w1 * rows[2 * t + 1, s]
            return carry

        lax.fori_loop(0, _C_TCH, body, 0)
        wcopy[c] = pltpu.async_copy(
            acc, out_hbm.at[pl.ds(tbase + c * _C_TCH, _C_TCH)], wsems[b])
    for c in sorted(wcopy):
        wcopy[c].wait()


def _sc_combine(og, slot3, wts3):
    return pl.kernel(
        _sc_combine_body,
        out_type=jax.ShapeDtypeStruct((N, D), jnp.float32),
        mesh=plsc.VectorSubcoreMesh(
            core_axis_name="c", subcore_axis_name="s",
            num_cores=NC, num_subcores=NS),
        scratch_types=[
            pltpu.VMEM((_C_NCH, 2 * _C_TCH), jnp.int32),
            pltpu.VMEM((_C_NCH, 2, _C_TCH), jnp.float32),
            pltpu.VMEM((2 * _C_TCH, D), jnp.float32),
            pltpu.VMEM((2 * _C_TCH, D), jnp.float32),
            pltpu.VMEM((_C_TCH, D), jnp.float32),
            pltpu.VMEM((_C_TCH, D), jnp.float32),
            pltpu.SemaphoreType.DMA,
            pltpu.SemaphoreType.DMA,
            pltpu.SemaphoreType.DMA,
            pltpu.SemaphoreType.DMA,
        ],
    )(og, slot3, wts3)


def _routing_meta(top2, wts2):
    i32 = jnp.int32
    e_flat = top2.reshape(-1)
    oh = (e_flat[:, None] == jnp.arange(E)[None, :]).astype(i32)
    csum = jnp.cumsum(oh, axis=0)
    counts = csum[-1]
    rank = jnp.take_along_axis(csum, e_flat[:, None], 1)[:, 0] - 1
    pc = ((counts + BLK_G - 1) // BLK_G) * BLK_G
    pe = jnp.cumsum(pc)
    po = pe - pc
    slot_of = (po[e_flat] + rank).astype(i32)
    be = jnp.minimum(
        jnp.sum((jnp.arange(G)[:, None] * BLK_G >= pe[None, :]).astype(i32),
                axis=1),
        E - 1).astype(i32)
    n_active = (pe[-1] // BLK_G).astype(i32)
    be = jnp.concatenate([be, n_active[None]])
    return slot_of, be


def kernel(x, W_gate, b_gate, W1, b1, W2, b2):
    x2 = jnp.asarray(x, jnp.float32).reshape(N, D)

    wg_pad = jnp.zeros((D, EPAD), jnp.float32).at[:, :E].set(W_gate)
    bg_pad = jnp.full((1, EPAD), -1e30, jnp.float32).at[0, :E].set(b_gate)
    noise = jax.random.normal(jax.random.key(42), (2, 2048, E)) * 0.01
    noise_pad = jnp.zeros((N, EPAD), jnp.float32).at[:, :E].set(
        noise.reshape(N, E))

    xn, idx128, wts128, aux128 = _gate_call(x2, wg_pad, bg_pad, noise_pad)
    top2 = idx128[:, :K]
    wts2 = wts128[:, :K]
    aux_loss = aux128[0, 0]

    slot_of, be = _routing_meta(top2, wts2)

    disp_slots = (slot_of.reshape(NW, _D_NCH, _D_TCH, K)
                  .transpose(0, 1, 3, 2).reshape(NW, 2 * _D_NCH, _D_TCH))
    xp = _sc_dispatch(xn, disp_slots)
    og = _gmm_call(be, xp, W1, b1.reshape(E, 1, F), W2, b2.reshape(E, 1, D))
    wts3 = (wts2.reshape(NW, _C_NCH, _C_TCH, K)
            .transpose(0, 1, 3, 2).reshape(NW, _C_NCH, 2, _C_TCH))
    out = _sc_combine(og, slot_of.reshape(NW, _C_NCH, 2 * _C_TCH), wts3)

    return (out.reshape(2, 2048, D), top2.reshape(2, 2048, K), aux_loss)
